# baseline (speedup 1.0000x reference)
import functools

import jax
import jax.numpy as jnp
from jax import lax
from jax.experimental import pallas as pl
from jax.experimental.pallas import tpu as pltpu
from jax.experimental.pallas import tpu_sc as plsc

OUT_SIZE = 1000000
N_EVENTS = 4194304
ACC = 1 << 20
LANES = 128
NC, NS = 2, 16
NW = NC * NS
EV_PER_W = N_EVENTS // NW
BLK = 8192
N_BLKS = EV_PER_W // BLK
STRIPE = ACC // NS
ACC_TOTAL = ACC + BLK


def _sc_scatter_partials(index, rate, starttime, endtime, tvec):
    mesh = plsc.VectorSubcoreMesh(core_axis_name="c", subcore_axis_name="s",
                                  num_cores=NC, num_subcores=NS)

    @functools.partial(
        pl.kernel,
        out_type=jax.ShapeDtypeStruct((NC, ACC), jnp.float32),
        mesh=mesh,
        scratch_types=dict(
            idxo_v=pltpu.VMEM((BLK,), jnp.int32),
            idx_v=pltpu.VMEM((BLK,), jnp.int32),
            rate_v=pltpu.VMEM((BLK,), jnp.float32),
            st_v=pltpu.VMEM((BLK,), jnp.float32),
            en_v=pltpu.VMEM((BLK,), jnp.float32),
            t_v=pltpu.VMEM((16,), jnp.float32),
            acc=pltpu.VMEM_SHARED((ACC_TOTAL,), jnp.float32),
            sem=pltpu.SemaphoreType.DMA,
        ),
    )
    def k(idx_h, rate_h, st_h, en_h, t_h, out_h, *, idxo_v, idx_v, rate_v,
          st_v, en_v, t_v, acc, sem):
        cid = lax.axis_index("c")
        sid = lax.axis_index("s")
        wid = cid * NS + sid

        @pl.loop(0, BLK // 16)
        def _(i):
            rate_v[pl.ds(i * 16, 16)] = jnp.zeros((16,), jnp.float32)

        for q in range(STRIPE // BLK):
            pltpu.sync_copy(rate_v, acc.at[pl.ds(sid * STRIPE + q * BLK, BLK)])

        pltpu.sync_copy(t_h, t_v)
        tv = t_v[...]
        plsc.subcore_barrier()

        for b in range(N_BLKS):
            base = wid * EV_PER_W + b * BLK
            pltpu.sync_copy(idx_h.at[pl.ds(base, BLK)], idx_v)
            pltpu.sync_copy(rate_h.at[pl.ds(base, BLK)], rate_v)
            pltpu.sync_copy(st_h.at[pl.ds(base, BLK)], st_v)
            pltpu.sync_copy(en_h.at[pl.ds(base, BLK)], en_v)

            lane = lax.iota(jnp.int32, 16)

            @pl.loop(0, BLK // 16)
            def _(i):
                sl = pl.ds(i * 16, 16)
                m = (st_v[sl] <= tv) & (tv < en_v[sl])
                dump = ACC + i * 16 + lane
                idxo_v[sl] = jnp.where(m, idx_v[sl], dump)

            pltpu.sync_copy(rate_v, acc.at[idxo_v], add=True)

        plsc.subcore_barrier()

        pltpu.sync_copy(acc.at[pl.ds(sid * STRIPE, STRIPE)],
                        out_h.at[cid, pl.ds(sid * STRIPE, STRIPE)])

    return k(index, rate, starttime, endtime, tvec)


def _tc_combine(partials):
    def body(p_ref, o_ref):
        o_ref[...] = p_ref[0] + p_ref[1]

    rows = ACC // LANES
    blk = 1024
    return pl.pallas_call(
        body,
        grid=(rows // blk,),
        in_specs=[pl.BlockSpec((NC, blk, LANES), lambda i: (0, i, 0))],
        out_specs=pl.BlockSpec((blk, LANES), lambda i: (i, 0)),
        out_shape=jax.ShapeDtypeStruct((rows, LANES), jnp.float32),
    )(partials)


def kernel(index, rate, starttime, endtime, t):
    tvec = jnp.full((16,), t, jnp.float32)
    partials = _sc_scatter_partials(index, rate, starttime, endtime, tvec)
    summed = _tc_combine(partials.reshape(NC, ACC // LANES, LANES))
    return summed.reshape(ACC)[:OUT_SIZE]


# speedup vs baseline: 1.3717x; 1.3717x over previous
import functools

import jax
import jax.numpy as jnp
from jax import lax
from jax.experimental import pallas as pl
from jax.experimental.pallas import tpu as pltpu
from jax.experimental.pallas import tpu_sc as plsc

OUT_SIZE = 1000000
N_EVENTS = 4194304
ACC = 1 << 20
LANES = 128
NC, NS = 2, 16
NW = NC * NS
EV_PER_W = N_EVENTS // NW
BLK = 4096
N_BLKS = EV_PER_W // BLK
STRIPE = ACC // NS
ACC_TOTAL = ACC + BLK


def _sc_scatter_partials(index, rate, starttime, endtime, tvec):
    mesh = plsc.VectorSubcoreMesh(core_axis_name="c", subcore_axis_name="s",
                                  num_cores=NC, num_subcores=NS)

    @functools.partial(
        pl.kernel,
        out_type=jax.ShapeDtypeStruct((NC, ACC), jnp.float32),
        mesh=mesh,
        scratch_types=dict(
            idx0=pltpu.VMEM((BLK,), jnp.int32),
            idx1=pltpu.VMEM((BLK,), jnp.int32),
            rate0=pltpu.VMEM((BLK,), jnp.float32),
            rate1=pltpu.VMEM((BLK,), jnp.float32),
            st0=pltpu.VMEM((BLK,), jnp.float32),
            st1=pltpu.VMEM((BLK,), jnp.float32),
            en0=pltpu.VMEM((BLK,), jnp.float32),
            en1=pltpu.VMEM((BLK,), jnp.float32),
            t_v=pltpu.VMEM((16,), jnp.float32),
            acc=pltpu.VMEM_SHARED((ACC_TOTAL,), jnp.float32),
            sem_in=pltpu.SemaphoreType.DMA,
            sem_sc=pltpu.SemaphoreType.DMA,
        ),
    )
    def k(idx_h, rate_h, st_h, en_h, t_h, out_h, *, idx0, idx1, rate0, rate1,
          st0, st1, en0, en1, t_v, acc, sem_in, sem_sc):
        cid = lax.axis_index("c")
        sid = lax.axis_index("s")
        wid = cid * NS + sid
        bufs = [(idx0, rate0, st0, en0), (idx1, rate1, st1, en1)]

        @pl.loop(0, BLK // 16)
        def _(i):
            rate0[pl.ds(i * 16, 16)] = jnp.zeros((16,), jnp.float32)

        zdescs = [
            pltpu.async_copy(rate0,
                             acc.at[pl.ds(sid * STRIPE + q * BLK, BLK)],
                             sem_in)
            for q in range(STRIPE // BLK)
        ]
        for d in zdescs:
            d.wait()

        pltpu.sync_copy(t_h, t_v)
        tv = t_v[...]
        lane = lax.iota(jnp.int32, 16)
        plsc.subcore_barrier()

        def stage(b):
            base = wid * EV_PER_W + b * BLK
            bi, br, bs, be = bufs[b % 2]
            return [
                pltpu.async_copy(idx_h.at[pl.ds(base, BLK)], bi, sem_in),
                pltpu.async_copy(rate_h.at[pl.ds(base, BLK)], br, sem_in),
                pltpu.async_copy(st_h.at[pl.ds(base, BLK)], bs, sem_in),
                pltpu.async_copy(en_h.at[pl.ds(base, BLK)], be, sem_in),
            ]

        in_flight = stage(0)
        sc_desc = None
        for b in range(N_BLKS):
            bi, br, bs, be = bufs[b % 2]
            for d in in_flight:
                d.wait()
            if sc_desc is not None:
                sc_desc.wait()
            if b + 1 < N_BLKS:
                in_flight = stage(b + 1)

            @pl.loop(0, BLK // 16, unroll=4)
            def _(i):
                sl = pl.ds(i * 16, 16)
                m = (bs[sl] <= tv) & (tv < be[sl])
                dump = ACC + i * 16 + lane
                bi[sl] = jnp.where(m, bi[sl], dump)

            sc_desc = pltpu.async_copy(br, acc.at[bi], sem_sc, add=True)
        sc_desc.wait()

        plsc.subcore_barrier()

        pltpu.sync_copy(acc.at[pl.ds(sid * STRIPE, STRIPE)],
                        out_h.at[cid, pl.ds(sid * STRIPE, STRIPE)])

    return k(index, rate, starttime, endtime, tvec)


def _tc_combine(partials):
    def body(p_ref, o_ref):
        o_ref[...] = p_ref[0] + p_ref[1]

    rows = ACC // LANES
    blk = 1024
    return pl.pallas_call(
        body,
        grid=(rows // blk,),
        in_specs=[pl.BlockSpec((NC, blk, LANES), lambda i: (0, i, 0))],
        out_specs=pl.BlockSpec((blk, LANES), lambda i: (i, 0)),
        out_shape=jax.ShapeDtypeStruct((rows, LANES), jnp.float32),
    )(partials)


def kernel(index, rate, starttime, endtime, t):
    tvec = jnp.full((16,), t, jnp.float32)
    partials = _sc_scatter_partials(index, rate, starttime, endtime, tvec)
    summed = _tc_combine(partials.reshape(NC, ACC // LANES, LANES))
    return summed.reshape(ACC)[:OUT_SIZE]


# speedup vs baseline: 1.8344x; 1.3373x over previous
import functools

import jax
import jax.numpy as jnp
from jax import lax
from jax.experimental import pallas as pl
from jax.experimental.pallas import tpu as pltpu
from jax.experimental.pallas import tpu_sc as plsc

OUT_SIZE = 1000000
N_EVENTS = 4194304
ACC = 1 << 20
LANES = 128
NC, NS = 2, 16
NW = NC * NS
EV_PER_W = N_EVENTS // NW
BLK = 4096
N_BLKS = EV_PER_W // BLK
STRIPE = ACC // NS
ACC_TOTAL = ACC + BLK


def _sc_scatter_partials(index, rate, starttime, endtime, tvec):
    mesh = plsc.VectorSubcoreMesh(core_axis_name="c", subcore_axis_name="s",
                                  num_cores=NC, num_subcores=NS)

    @functools.partial(
        pl.kernel,
        out_type=jax.ShapeDtypeStruct((NC, ACC), jnp.float32),
        mesh=mesh,
        scratch_types=dict(
            idx0=pltpu.VMEM((BLK,), jnp.int32),
            idx1=pltpu.VMEM((BLK,), jnp.int32),
            idx2=pltpu.VMEM((BLK,), jnp.int32),
            rate0=pltpu.VMEM((BLK,), jnp.float32),
            rate1=pltpu.VMEM((BLK,), jnp.float32),
            rate2=pltpu.VMEM((BLK,), jnp.float32),
            st0=pltpu.VMEM((BLK,), jnp.float32),
            st1=pltpu.VMEM((BLK,), jnp.float32),
            st2=pltpu.VMEM((BLK,), jnp.float32),
            en0=pltpu.VMEM((BLK,), jnp.float32),
            en1=pltpu.VMEM((BLK,), jnp.float32),
            en2=pltpu.VMEM((BLK,), jnp.float32),
            t_v=pltpu.VMEM((16,), jnp.float32),
            acc=pltpu.VMEM_SHARED((ACC_TOTAL,), jnp.float32),
            sem_in=pltpu.SemaphoreType.DMA,
            sem_sc=pltpu.SemaphoreType.DMA,
        ),
    )
    def k(idx_h, rate_h, st_h, en_h, t_h, out_h, *, idx0, idx1, idx2,
          rate0, rate1, rate2, st0, st1, st2, en0, en1, en2, t_v, acc,
          sem_in, sem_sc):
        cid = lax.axis_index("c")
        sid = lax.axis_index("s")
        wid = cid * NS + sid
        bufs = [(idx0, rate0, st0, en0), (idx1, rate1, st1, en1),
                (idx2, rate2, st2, en2)]

        @pl.loop(0, BLK // 16)
        def _(i):
            rate0[pl.ds(i * 16, 16)] = jnp.zeros((16,), jnp.float32)

        zdescs = [
            pltpu.async_copy(rate0,
                             acc.at[pl.ds(sid * STRIPE + q * BLK, BLK)],
                             sem_in)
            for q in range(STRIPE // BLK)
        ]
        for d in zdescs:
            d.wait()

        pltpu.sync_copy(t_h, t_v)
        tv = t_v[...]
        lane = lax.iota(jnp.int32, 16)
        plsc.subcore_barrier()

        def stage(b):
            base = wid * EV_PER_W + b * BLK
            bi, br, bs, be = bufs[b % 3]
            return [
                pltpu.async_copy(idx_h.at[pl.ds(base, BLK)], bi, sem_in),
                pltpu.async_copy(rate_h.at[pl.ds(base, BLK)], br, sem_in),
                pltpu.async_copy(st_h.at[pl.ds(base, BLK)], bs, sem_in),
                pltpu.async_copy(en_h.at[pl.ds(base, BLK)], be, sem_in),
            ]

        in_flight = stage(0)
        sc_descs = [None] * N_BLKS
        for b in range(N_BLKS):
            bi, br, bs, be = bufs[b % 3]
            for d in in_flight:
                d.wait()
            if b >= 2:
                sc_descs[b - 2].wait()
            if b + 1 < N_BLKS:
                in_flight = stage(b + 1)

            @pl.loop(0, BLK // 16, unroll=4)
            def _(i):
                sl = pl.ds(i * 16, 16)
                m = (bs[sl] <= tv) & (tv < be[sl])
                dump = ACC + i * 16 + lane
                bi[sl] = jnp.where(m, bi[sl], dump)

            sc_descs[b] = pltpu.async_copy(br, acc.at[bi], sem_sc, add=True)
        sc_descs[N_BLKS - 2].wait()
        sc_descs[N_BLKS - 1].wait()

        plsc.subcore_barrier()

        pltpu.sync_copy(acc.at[pl.ds(sid * STRIPE, STRIPE)],
                        out_h.at[cid, pl.ds(sid * STRIPE, STRIPE)])

    return k(index, rate, starttime, endtime, tvec)


def _tc_combine(partials):
    def body(p_ref, o_ref):
        o_ref[...] = p_ref[0] + p_ref[1]

    rows = ACC // LANES
    blk = 1024
    return pl.pallas_call(
        body,
        grid=(rows // blk,),
        in_specs=[pl.BlockSpec((NC, blk, LANES), lambda i: (0, i, 0))],
        out_specs=pl.BlockSpec((blk, LANES), lambda i: (i, 0)),
        out_shape=jax.ShapeDtypeStruct((rows, LANES), jnp.float32),
    )(partials)


def kernel(index, rate, starttime, endtime, t):
    tvec = jnp.full((16,), t, jnp.float32)
    partials = _sc_scatter_partials(index, rate, starttime, endtime, tvec)
    summed = _tc_combine(partials.reshape(NC, ACC // LANES, LANES))
    return summed.reshape(ACC)[:OUT_SIZE]


# speedup vs baseline: 2.1526x; 1.1735x over previous
import functools

import jax
import jax.numpy as jnp
from jax import lax
from jax.experimental import pallas as pl
from jax.experimental.pallas import tpu as pltpu
from jax.experimental.pallas import tpu_sc as plsc

OUT_SIZE = 1000000
N_EVENTS = 4194304
ACC = 1 << 20
LANES = 128
NC, NS = 2, 16
NW = NC * NS
EV_PER_W = N_EVENTS // NW
BLK = 4096
N_BLKS = EV_PER_W // BLK
STRIPE = ACC // NS
ACC_TOTAL = ACC + BLK


def _sc_scatter_partials(index, rate, starttime, endtime, tvec):
    mesh = plsc.VectorSubcoreMesh(core_axis_name="c", subcore_axis_name="s",
                                  num_cores=NC, num_subcores=NS)

    @functools.partial(
        pl.kernel,
        out_type=jax.ShapeDtypeStruct((NC, ACC), jnp.float32),
        mesh=mesh,
        scratch_types=dict(
            idx0=pltpu.VMEM((BLK,), jnp.int32),
            idx1=pltpu.VMEM((BLK,), jnp.int32),
            idx2=pltpu.VMEM((BLK,), jnp.int32),
            rate0=pltpu.VMEM((BLK,), jnp.float32),
            rate1=pltpu.VMEM((BLK,), jnp.float32),
            rate2=pltpu.VMEM((BLK,), jnp.float32),
            st0=pltpu.VMEM((BLK,), jnp.float32),
            st1=pltpu.VMEM((BLK,), jnp.float32),
            st2=pltpu.VMEM((BLK,), jnp.float32),
            en0=pltpu.VMEM((BLK,), jnp.float32),
            en1=pltpu.VMEM((BLK,), jnp.float32),
            en2=pltpu.VMEM((BLK,), jnp.float32),
            t_v=pltpu.VMEM((16,), jnp.float32),
            acc=pltpu.VMEM_SHARED((ACC_TOTAL,), jnp.float32),
            sem_in=pltpu.SemaphoreType.DMA,
            sem_sc=pltpu.SemaphoreType.DMA,
        ),
    )
    def k(idx_h, rate_h, st_h, en_h, t_h, out_h, *, idx0, idx1, idx2,
          rate0, rate1, rate2, st0, st1, st2, en0, en1, en2, t_v, acc,
          sem_in, sem_sc):
        cid = lax.axis_index("c")
        sid = lax.axis_index("s")
        wid = cid * NS + sid
        bufs = [(idx0, rate0, st0, en0), (idx1, rate1, st1, en1),
                (idx2, rate2, st2, en2)]

        @pl.loop(0, BLK // 16)
        def _(i):
            rate0[pl.ds(i * 16, 16)] = jnp.zeros((16,), jnp.float32)

        zdescs = [
            pltpu.async_copy(rate0,
                             acc.at[pl.ds(sid * STRIPE + q * BLK, BLK)],
                             sem_in)
            for q in range(STRIPE // BLK)
        ]
        for d in zdescs:
            d.wait()

        pltpu.sync_copy(t_h, t_v)
        tv = t_v[...]
        lane = lax.iota(jnp.int32, 16)
        plsc.subcore_barrier()

        def stage(b):
            base = wid * EV_PER_W + b * BLK
            bi, br, bs, be = bufs[b % 3]
            return [
                pltpu.async_copy(idx_h.at[pl.ds(base, BLK)], bi, sem_in),
                pltpu.async_copy(rate_h.at[pl.ds(base, BLK)], br, sem_in),
                pltpu.async_copy(st_h.at[pl.ds(base, BLK)], bs, sem_in),
                pltpu.async_copy(en_h.at[pl.ds(base, BLK)], be, sem_in),
            ]

        in_flight = stage(0)
        sc_descs = [None] * N_BLKS
        for b in range(N_BLKS):
            bi, br, bs, be = bufs[b % 3]
            for d in in_flight:
                d.wait()
            if b >= 2:
                sc_descs[b - 2].wait()
            if b + 1 < N_BLKS:
                in_flight = stage(b + 1)

            @pl.loop(0, BLK // 16, unroll=4)
            def _(i):
                sl = pl.ds(i * 16, 16)
                m = (bs[sl] <= tv) & (tv < be[sl])
                dump = ACC + i * 16 + lane
                bi[sl] = jnp.where(m, bi[sl], dump)

            sc_descs[b] = pltpu.async_copy(br, acc.at[bi], sem_sc, add=True)
        sc_descs[N_BLKS - 2].wait()
        sc_descs[N_BLKS - 1].wait()

        plsc.subcore_barrier()

        pltpu.sync_copy(acc.at[pl.ds(sid * STRIPE, STRIPE)],
                        out_h.at[cid, pl.ds(sid * STRIPE, STRIPE)])

    return k(index, rate, starttime, endtime, tvec)


def _tc_combine(partials):
    def body(p_ref, o_ref):
        o_ref[...] = p_ref[0] + p_ref[1]

    blk = ACC // 8
    return pl.pallas_call(
        body,
        grid=(8,),
        in_specs=[pl.BlockSpec((NC, blk), lambda i: (0, i))],
        out_specs=pl.BlockSpec((blk,), lambda i: (i,)),
        out_shape=jax.ShapeDtypeStruct((OUT_SIZE,), jnp.float32),
    )(partials)


def kernel(index, rate, starttime, endtime, t):
    tvec = jnp.full((16,), t, jnp.float32)
    partials = _sc_scatter_partials(index, rate, starttime, endtime, tvec)
    return _tc_combine(partials)
